# Initial kernel scaffold; baseline (speedup 1.0000x reference)
import functools

import jax
import jax.numpy as jnp
from jax import lax
from jax.experimental import pallas as pl
from jax.experimental.pallas import tpu as pltpu
from jax.experimental.pallas import tpu_sc as plsc

N_CLOTH = 8192
N_VERTS = 8192
N_FACES = 8192

WEIGHT_START = 1.0
WEIGHT_MAX = 5000.0
START_RAMPUP = 50000
N_RAMPUP = 100000
EPS = 1e-3

NC = 2
NS = 16
NW = NC * NS
L = 16

FPW = N_FACES // NW
PPW = N_CLOTH // NW

_MESH = dict(core_axis_name="c", subcore_axis_name="s", num_cores=NC,
             num_subcores=NS)


def _wid():
    return lax.axis_index("s") * NC + lax.axis_index("c")


def _stage_faces(opx, opy, opz, otx, oty, otz, f0, f1, f2, interpret=False):
    fvec = jax.ShapeDtypeStruct((N_FACES,), jnp.float32)

    @functools.partial(
        pl.kernel,
        out_type=[fvec] * 9,
        mesh=plsc.VectorSubcoreMesh(**_MESH),
        scratch_types=[
            pltpu.VMEM((N_VERTS,), jnp.float32),
            pltpu.VMEM((N_VERTS,), jnp.float32),
            pltpu.VMEM((N_VERTS,), jnp.float32),
            pltpu.VMEM((N_VERTS,), jnp.float32),
            pltpu.VMEM((N_VERTS,), jnp.float32),
            pltpu.VMEM((N_VERTS,), jnp.float32),
            pltpu.VMEM((FPW,), jnp.int32),
            pltpu.VMEM((FPW,), jnp.int32),
            pltpu.VMEM((FPW,), jnp.int32),
        ] + [pltpu.VMEM((FPW,), jnp.float32)] * 9,
        interpret=interpret,
    )
    def body(opx_h, opy_h, opz_h, otx_h, oty_h, otz_h, f0_h, f1_h, f2_h,
             m2fx_h, m2fy_h, m2fz_h, fsq_h, nx_h, ny_h, nz_h, nsq_h, braw_h,
             opx_v, opy_v, opz_v, otx_v, oty_v, otz_v, f0_v, f1_v, f2_v,
             m2fx_v, m2fy_v, m2fz_v, fsq_v, nx_v, ny_v, nz_v, nsq_v, braw_v):
        base = _wid() * FPW
        pltpu.sync_copy(opx_h, opx_v)
        pltpu.sync_copy(opy_h, opy_v)
        pltpu.sync_copy(opz_h, opz_v)
        pltpu.sync_copy(otx_h, otx_v)
        pltpu.sync_copy(oty_h, oty_v)
        pltpu.sync_copy(otz_h, otz_v)
        pltpu.sync_copy(f0_h.at[pl.ds(base, FPW)], f0_v)
        pltpu.sync_copy(f1_h.at[pl.ds(base, FPW)], f1_v)
        pltpu.sync_copy(f2_h.at[pl.ds(base, FPW)], f2_v)
        for i in range(FPW // L):
            sl = pl.ds(i * L, L)
            i0 = f0_v[sl]
            i1 = f1_v[sl]
            i2 = f2_v[sl]
            ax = plsc.load_gather(opx_v, [i0])
            bx = plsc.load_gather(opx_v, [i1])
            cx = plsc.load_gather(opx_v, [i2])
            ay = plsc.load_gather(opy_v, [i0])
            by = plsc.load_gather(opy_v, [i1])
            cy = plsc.load_gather(opy_v, [i2])
            az = plsc.load_gather(opz_v, [i0])
            bz = plsc.load_gather(opz_v, [i1])
            cz = plsc.load_gather(opz_v, [i2])
            third = jnp.float32(1.0 / 3.0)
            fx = (ax + bx + cx) * third
            fy = (ay + by + cy) * third
            fz = (az + bz + cz) * third
            m2fx_v[sl] = -2.0 * fx
            m2fy_v[sl] = -2.0 * fy
            m2fz_v[sl] = -2.0 * fz
            fsq_v[sl] = fx * fx + fy * fy + fz * fz
            tax = plsc.load_gather(otx_v, [i0])
            tbx = plsc.load_gather(otx_v, [i1])
            tcx = plsc.load_gather(otx_v, [i2])
            tay = plsc.load_gather(oty_v, [i0])
            tby = plsc.load_gather(oty_v, [i1])
            tcy = plsc.load_gather(oty_v, [i2])
            taz = plsc.load_gather(otz_v, [i0])
            tbz = plsc.load_gather(otz_v, [i1])
            tcz = plsc.load_gather(otz_v, [i2])
            px = (tax + tbx + tcx) * third
            py = (tay + tby + tcy) * third
            pz = (taz + tbz + tcz) * third
            e1x = tbx - tax
            e1y = tby - tay
            e1z = tbz - taz
            e2x = tcx - tax
            e2y = tcy - tay
            e2z = tcz - taz
            nx = e1y * e2z - e1z * e2y
            ny = e1z * e2x - e1x * e2z
            nz = e1x * e2y - e1y * e2x
            nx_v[sl] = nx
            ny_v[sl] = ny
            nz_v[sl] = nz
            nsq_v[sl] = nx * nx + ny * ny + nz * nz
            braw_v[sl] = px * nx + py * ny + pz * nz
        out_sl = pl.ds(base, FPW)
        pltpu.sync_copy(m2fx_v, m2fx_h.at[out_sl])
        pltpu.sync_copy(m2fy_v, m2fy_h.at[out_sl])
        pltpu.sync_copy(m2fz_v, m2fz_h.at[out_sl])
        pltpu.sync_copy(fsq_v, fsq_h.at[out_sl])
        pltpu.sync_copy(nx_v, nx_h.at[out_sl])
        pltpu.sync_copy(ny_v, ny_h.at[out_sl])
        pltpu.sync_copy(nz_v, nz_h.at[out_sl])
        pltpu.sync_copy(nsq_v, nsq_h.at[out_sl])
        pltpu.sync_copy(braw_v, braw_h.at[out_sl])

    return body(opx, opy, opz, otx, oty, otz, f0, f1, f2)


_ROWS = 256
_CHUNK = 1024


def _stage_argmin(c4, fp, interpret=False):
    def body(c4_ref, fp_ref, idx_ref):
        c = c4_ref[...]
        acc_min = jnp.full((_ROWS, 1), jnp.inf, dtype=jnp.float32)
        acc_idx = jnp.zeros((_ROWS, 1), dtype=jnp.int32)
        for jt in range(N_FACES // _CHUNK):
            f = fp_ref[:, jt * _CHUNK:(jt + 1) * _CHUNK]
            s = lax.dot_general(
                c, f, (((1,), (0,)), ((), ())),
                preferred_element_type=jnp.float32,
                precision=lax.Precision.HIGHEST,
            )
            tmin = jnp.min(s, axis=1, keepdims=True)
            jglob = lax.broadcasted_iota(jnp.int32, (_ROWS, _CHUNK), 1) \
                + jnp.int32(jt * _CHUNK)
            tidx = jnp.min(
                jnp.where(s <= tmin, jglob, jnp.int32(2**31 - 1)),
                axis=1, keepdims=True)
            upd = tmin < acc_min
            acc_idx = jnp.where(upd, tidx, acc_idx)
            acc_min = jnp.where(upd, tmin, acc_min)
        idx_ref[...] = acc_idx

    return pl.pallas_call(
        body,
        grid=(N_CLOTH // _ROWS,),
        in_specs=[
            pl.BlockSpec((_ROWS, 8), lambda i: (i, 0)),
            pl.BlockSpec((8, N_FACES), lambda i: (0, 0)),
        ],
        out_specs=pl.BlockSpec((_ROWS, 1), lambda i: (i, 0)),
        out_shape=jax.ShapeDtypeStruct((N_CLOTH, 1), jnp.int32),
        interpret=interpret,
    )(c4, fp)


def _stage_loss(nn_idx, nx, ny, nz, nsq, braw, ppx, ppy, ppz,
                interpret=False):
    @functools.partial(
        pl.kernel,
        out_type=jax.ShapeDtypeStruct((NW, L), jnp.float32),
        mesh=plsc.VectorSubcoreMesh(**_MESH),
        scratch_types=[
            pltpu.VMEM((N_FACES,), jnp.float32),
            pltpu.VMEM((N_FACES,), jnp.float32),
            pltpu.VMEM((N_FACES,), jnp.float32),
            pltpu.VMEM((N_FACES,), jnp.float32),
            pltpu.VMEM((N_FACES,), jnp.float32),
            pltpu.VMEM((PPW,), jnp.int32),
            pltpu.VMEM((PPW,), jnp.float32),
            pltpu.VMEM((PPW,), jnp.float32),
            pltpu.VMEM((PPW,), jnp.float32),
            pltpu.VMEM((L,), jnp.float32),
        ],
        interpret=interpret,
    )
    def body(idx_h, nx_h, ny_h, nz_h, nsq_h, braw_h, ppx_h, ppy_h, ppz_h,
             out_h,
             nx_v, ny_v, nz_v, nsq_v, braw_v, idx_v, ppx_v, ppy_v, ppz_v,
             acc_v):
        wid = _wid()
        base = wid * PPW
        pltpu.sync_copy(nx_h, nx_v)
        pltpu.sync_copy(ny_h, ny_v)
        pltpu.sync_copy(nz_h, nz_v)
        pltpu.sync_copy(nsq_h, nsq_v)
        pltpu.sync_copy(braw_h, braw_v)
        pltpu.sync_copy(idx_h.at[pl.ds(base, PPW)], idx_v)
        pltpu.sync_copy(ppx_h.at[pl.ds(base, PPW)], ppx_v)
        pltpu.sync_copy(ppy_h.at[pl.ds(base, PPW)], ppy_v)
        pltpu.sync_copy(ppz_h.at[pl.ds(base, PPW)], ppz_v)
        acc = jnp.zeros((L,), jnp.float32)
        for i in range(PPW // L):
            sl = pl.ds(i * L, L)
            ii = idx_v[sl]
            gx = plsc.load_gather(nx_v, [ii])
            gy = plsc.load_gather(ny_v, [ii])
            gz = plsc.load_gather(nz_v, [ii])
            gq = plsc.load_gather(nsq_v, [ii])
            gb = plsc.load_gather(braw_v, [ii])
            draw = ppx_v[sl] * gx + ppy_v[sl] * gy + ppz_v[sl] * gz - gb
            xc = jnp.maximum(gq, jnp.float32(1e-36))
            y = plsc.bitcast(
                jnp.int32(0x5F3759DF) - (plsc.bitcast(xc, jnp.int32) >> 1),
                jnp.float32)
            for _ in range(3):
                y = y * (1.5 - 0.5 * xc * y * y)
            snorm = gq * y
            dist = draw / (snorm + jnp.float32(1e-12))
            t = jnp.maximum(jnp.float32(EPS) - dist, 0.0)
            acc = acc + t * t * t
        acc_v[...] = acc
        pltpu.sync_copy(acc_v, out_h.at[wid])

    return body(nn_idx, nx, ny, nz, nsq, braw, ppx, ppy, ppz)


def kernel(cloth_pos, cloth_pred_pos, obstacle_pos, obstacle_target_pos,
           obstacle_faces, iter_num):
    opx, opy, opz = (obstacle_pos[:, k] for k in range(3))
    otx, oty, otz = (obstacle_target_pos[:, k] for k in range(3))
    f0, f1, f2 = (obstacle_faces[:, k] for k in range(3))

    m2fx, m2fy, m2fz, fsq, nx, ny, nz, nsq, braw = _stage_faces(
        opx, opy, opz, otx, oty, otz, f0, f1, f2)

    c4 = jnp.concatenate(
        [cloth_pos,
         jnp.ones((N_CLOTH, 1), jnp.float32),
         jnp.zeros((N_CLOTH, 4), jnp.float32)], axis=1)
    fp = jnp.concatenate(
        [m2fx[None], m2fy[None], m2fz[None], fsq[None],
         jnp.zeros((4, N_FACES), jnp.float32)], axis=0)
    nn_idx = _stage_argmin(c4, fp)[:, 0]

    ppx, ppy, ppz = (cloth_pred_pos[:, k] for k in range(3))
    partials = _stage_loss(nn_idx, nx, ny, nz, nsq, braw, ppx, ppy, ppz)

    it = jnp.maximum(iter_num - START_RAMPUP, 0)
    progress = jnp.minimum(it / N_RAMPUP, 1.0)
    weight = (WEIGHT_START + (WEIGHT_MAX - WEIGHT_START) * progress)
    return jnp.sum(partials) * weight.astype(jnp.float32)


# speedup vs baseline: 2.8720x; 2.8720x over previous
import functools

import jax
import jax.numpy as jnp
from jax import lax
from jax.experimental import pallas as pl
from jax.experimental.pallas import tpu as pltpu
from jax.experimental.pallas import tpu_sc as plsc

N_CLOTH = 8192
N_VERTS = 8192
N_FACES = 8192

WEIGHT_START = 1.0
WEIGHT_MAX = 5000.0
START_RAMPUP = 50000
N_RAMPUP = 100000
EPS = 1e-3

NC = 2
NS = 16
NW = NC * NS
L = 16

FPW = N_FACES // NW
PPW = N_CLOTH // NW

_MESH = dict(core_axis_name="c", subcore_axis_name="s", num_cores=NC,
             num_subcores=NS)


def _wid():
    return lax.axis_index("s") * NC + lax.axis_index("c")


def _stage_faces(opx, opy, opz, otx, oty, otz, f0, f1, f2, interpret=False):
    fvec = jax.ShapeDtypeStruct((N_FACES,), jnp.float32)

    @functools.partial(
        pl.kernel,
        out_type=[fvec] * 9,
        mesh=plsc.VectorSubcoreMesh(**_MESH),
        scratch_types=[
            pltpu.VMEM((N_VERTS,), jnp.float32),
            pltpu.VMEM((N_VERTS,), jnp.float32),
            pltpu.VMEM((N_VERTS,), jnp.float32),
            pltpu.VMEM((N_VERTS,), jnp.float32),
            pltpu.VMEM((N_VERTS,), jnp.float32),
            pltpu.VMEM((N_VERTS,), jnp.float32),
            pltpu.VMEM((FPW,), jnp.int32),
            pltpu.VMEM((FPW,), jnp.int32),
            pltpu.VMEM((FPW,), jnp.int32),
        ] + [pltpu.VMEM((FPW,), jnp.float32)] * 9,
        compiler_params=pltpu.CompilerParams(needs_layout_passes=False),
        interpret=interpret,
    )
    def body(opx_h, opy_h, opz_h, otx_h, oty_h, otz_h, f0_h, f1_h, f2_h,
             fcx_h, fcy_h, fcz_h, fsq_h, nx_h, ny_h, nz_h, nsq_h, braw_h,
             opx_v, opy_v, opz_v, otx_v, oty_v, otz_v, f0_v, f1_v, f2_v,
             fcx_v, fcy_v, fcz_v, fsq_v, nx_v, ny_v, nz_v, nsq_v, braw_v):
        base = _wid() * FPW
        pltpu.sync_copy(opx_h, opx_v)
        pltpu.sync_copy(opy_h, opy_v)
        pltpu.sync_copy(opz_h, opz_v)
        pltpu.sync_copy(otx_h, otx_v)
        pltpu.sync_copy(oty_h, oty_v)
        pltpu.sync_copy(otz_h, otz_v)
        pltpu.sync_copy(f0_h.at[pl.ds(base, FPW)], f0_v)
        pltpu.sync_copy(f1_h.at[pl.ds(base, FPW)], f1_v)
        pltpu.sync_copy(f2_h.at[pl.ds(base, FPW)], f2_v)
        for i in range(FPW // L):
            sl = pl.ds(i * L, L)
            i0 = f0_v[sl]
            i1 = f1_v[sl]
            i2 = f2_v[sl]
            ax = plsc.load_gather(opx_v, [i0])
            bx = plsc.load_gather(opx_v, [i1])
            cx = plsc.load_gather(opx_v, [i2])
            ay = plsc.load_gather(opy_v, [i0])
            by = plsc.load_gather(opy_v, [i1])
            cy = plsc.load_gather(opy_v, [i2])
            az = plsc.load_gather(opz_v, [i0])
            bz = plsc.load_gather(opz_v, [i1])
            cz = plsc.load_gather(opz_v, [i2])
            third = jnp.float32(1.0 / 3.0)
            fx = (ax + bx + cx) * third
            fy = (ay + by + cy) * third
            fz = (az + bz + cz) * third
            fcx_v[sl] = fx
            fcy_v[sl] = fy
            fcz_v[sl] = fz
            fsq_v[sl] = fx * fx + fy * fy + fz * fz
            tax = plsc.load_gather(otx_v, [i0])
            tbx = plsc.load_gather(otx_v, [i1])
            tcx = plsc.load_gather(otx_v, [i2])
            tay = plsc.load_gather(oty_v, [i0])
            tby = plsc.load_gather(oty_v, [i1])
            tcy = plsc.load_gather(oty_v, [i2])
            taz = plsc.load_gather(otz_v, [i0])
            tbz = plsc.load_gather(otz_v, [i1])
            tcz = plsc.load_gather(otz_v, [i2])
            px = (tax + tbx + tcx) * third
            py = (tay + tby + tcy) * third
            pz = (taz + tbz + tcz) * third
            e1x = tbx - tax
            e1y = tby - tay
            e1z = tbz - taz
            e2x = tcx - tax
            e2y = tcy - tay
            e2z = tcz - taz
            nx = e1y * e2z - e1z * e2y
            ny = e1z * e2x - e1x * e2z
            nz = e1x * e2y - e1y * e2x
            nx_v[sl] = nx
            ny_v[sl] = ny
            nz_v[sl] = nz
            nsq_v[sl] = nx * nx + ny * ny + nz * nz
            braw_v[sl] = px * nx + py * ny + pz * nz
        out_sl = pl.ds(base, FPW)
        pltpu.sync_copy(fcx_v, fcx_h.at[out_sl])
        pltpu.sync_copy(fcy_v, fcy_h.at[out_sl])
        pltpu.sync_copy(fcz_v, fcz_h.at[out_sl])
        pltpu.sync_copy(fsq_v, fsq_h.at[out_sl])
        pltpu.sync_copy(nx_v, nx_h.at[out_sl])
        pltpu.sync_copy(ny_v, ny_h.at[out_sl])
        pltpu.sync_copy(nz_v, nz_h.at[out_sl])
        pltpu.sync_copy(nsq_v, nsq_h.at[out_sl])
        pltpu.sync_copy(braw_v, braw_h.at[out_sl])

    return body(opx, opy, opz, otx, oty, otz, f0, f1, f2)


_ROWS = 256
_CHUNK = 1024


def _stage_argmin(c4, fp, interpret=False):
    def body(c4_ref, fp_ref, idx_ref):
        c = c4_ref[...]
        cb = c.astype(jnp.bfloat16)
        csq = (c[:, 0:1] * c[:, 0:1] + c[:, 1:2] * c[:, 1:2]
               + c[:, 2:3] * c[:, 2:3])
        acc_min = jnp.full((_ROWS, 1), jnp.inf, dtype=jnp.float32)
        acc_idx = jnp.zeros((_ROWS, 1), dtype=jnp.int32)
        for jt in range(N_FACES // _CHUNK):
            f = fp_ref[:, jt * _CHUNK:(jt + 1) * _CHUNK]
            fsq = f[3:4, :]
            mm = lax.dot_general(
                cb, f.astype(jnp.bfloat16), (((1,), (0,)), ((), ())),
                preferred_element_type=jnp.float32,
            )
            s = (csq + fsq) - 2.0 * mm
            tmin = jnp.min(s, axis=1, keepdims=True)
            jglob = lax.broadcasted_iota(jnp.int32, (_ROWS, _CHUNK), 1) \
                + jnp.int32(jt * _CHUNK)
            tidx = jnp.min(
                jnp.where(s <= tmin, jglob, jnp.int32(2**31 - 1)),
                axis=1, keepdims=True)
            upd = tmin < acc_min
            acc_idx = jnp.where(upd, tidx, acc_idx)
            acc_min = jnp.where(upd, tmin, acc_min)
        idx_ref[...] = acc_idx

    return pl.pallas_call(
        body,
        grid=(N_CLOTH // _ROWS,),
        in_specs=[
            pl.BlockSpec((_ROWS, 8), lambda i: (i, 0)),
            pl.BlockSpec((8, N_FACES), lambda i: (0, 0)),
        ],
        out_specs=pl.BlockSpec((_ROWS, 1), lambda i: (i, 0)),
        out_shape=jax.ShapeDtypeStruct((N_CLOTH, 1), jnp.int32),
        interpret=interpret,
    )(c4, fp)


def _stage_loss(nn_idx, nx, ny, nz, nsq, braw, ppx, ppy, ppz,
                interpret=False):
    @functools.partial(
        pl.kernel,
        out_type=jax.ShapeDtypeStruct((NW, L), jnp.float32),
        mesh=plsc.VectorSubcoreMesh(**_MESH),
        scratch_types=[
            pltpu.VMEM((N_FACES,), jnp.float32),
            pltpu.VMEM((N_FACES,), jnp.float32),
            pltpu.VMEM((N_FACES,), jnp.float32),
            pltpu.VMEM((N_FACES,), jnp.float32),
            pltpu.VMEM((N_FACES,), jnp.float32),
            pltpu.VMEM((PPW,), jnp.int32),
            pltpu.VMEM((PPW,), jnp.float32),
            pltpu.VMEM((PPW,), jnp.float32),
            pltpu.VMEM((PPW,), jnp.float32),
            pltpu.VMEM((L,), jnp.float32),
        ],
        compiler_params=pltpu.CompilerParams(needs_layout_passes=False),
        interpret=interpret,
    )
    def body(idx_h, nx_h, ny_h, nz_h, nsq_h, braw_h, ppx_h, ppy_h, ppz_h,
             out_h,
             nx_v, ny_v, nz_v, nsq_v, braw_v, idx_v, ppx_v, ppy_v, ppz_v,
             acc_v):
        wid = _wid()
        base = wid * PPW
        pltpu.sync_copy(nx_h, nx_v)
        pltpu.sync_copy(ny_h, ny_v)
        pltpu.sync_copy(nz_h, nz_v)
        pltpu.sync_copy(nsq_h, nsq_v)
        pltpu.sync_copy(braw_h, braw_v)
        pltpu.sync_copy(idx_h.at[pl.ds(base, PPW)], idx_v)
        pltpu.sync_copy(ppx_h.at[pl.ds(base, PPW)], ppx_v)
        pltpu.sync_copy(ppy_h.at[pl.ds(base, PPW)], ppy_v)
        pltpu.sync_copy(ppz_h.at[pl.ds(base, PPW)], ppz_v)
        acc = jnp.zeros((L,), jnp.float32)
        for i in range(PPW // L):
            sl = pl.ds(i * L, L)
            ii = idx_v[sl]
            gx = plsc.load_gather(nx_v, [ii])
            gy = plsc.load_gather(ny_v, [ii])
            gz = plsc.load_gather(nz_v, [ii])
            gq = plsc.load_gather(nsq_v, [ii])
            gb = plsc.load_gather(braw_v, [ii])
            draw = ppx_v[sl] * gx + ppy_v[sl] * gy + ppz_v[sl] * gz - gb
            xc = jnp.maximum(gq, jnp.float32(1e-36))
            y = plsc.bitcast(
                jnp.int32(0x5F3759DF) - (plsc.bitcast(xc, jnp.int32) >> 1),
                jnp.float32)
            for _ in range(3):
                y = y * (1.5 - 0.5 * xc * y * y)
            snorm = gq * y
            dist = draw / (snorm + jnp.float32(1e-12))
            t = jnp.maximum(jnp.float32(EPS) - dist, 0.0)
            acc = acc + t * t * t
        acc_v[...] = acc
        pltpu.sync_copy(acc_v, out_h.at[wid])

    return body(nn_idx, nx, ny, nz, nsq, braw, ppx, ppy, ppz)


def kernel(cloth_pos, cloth_pred_pos, obstacle_pos, obstacle_target_pos,
           obstacle_faces, iter_num):
    opx, opy, opz = (obstacle_pos[:, k] for k in range(3))
    otx, oty, otz = (obstacle_target_pos[:, k] for k in range(3))
    f0, f1, f2 = (obstacle_faces[:, k] for k in range(3))

    fcx, fcy, fcz, fsq, nx, ny, nz, nsq, braw = _stage_faces(
        opx, opy, opz, otx, oty, otz, f0, f1, f2)

    c4 = jnp.concatenate(
        [cloth_pos, jnp.zeros((N_CLOTH, 5), jnp.float32)], axis=1)
    fp = jnp.concatenate(
        [fcx[None], fcy[None], fcz[None], fsq[None],
         jnp.zeros((4, N_FACES), jnp.float32)], axis=0)
    nn_idx = _stage_argmin(c4, fp)[:, 0]

    ppx, ppy, ppz = (cloth_pred_pos[:, k] for k in range(3))
    partials = _stage_loss(nn_idx, nx, ny, nz, nsq, braw, ppx, ppy, ppz)

    it = jnp.maximum(iter_num - START_RAMPUP, 0)
    progress = jnp.minimum(it / N_RAMPUP, 1.0)
    weight = (WEIGHT_START + (WEIGHT_MAX - WEIGHT_START) * progress)
    return jnp.sum(partials) * weight.astype(jnp.float32)


# speedup vs baseline: 3.7010x; 1.2887x over previous
import functools

import jax
import jax.numpy as jnp
from jax import lax
from jax.experimental import pallas as pl
from jax.experimental.pallas import tpu as pltpu
from jax.experimental.pallas import tpu_sc as plsc

N_CLOTH = 8192
N_VERTS = 8192
N_FACES = 8192

WEIGHT_START = 1.0
WEIGHT_MAX = 5000.0
START_RAMPUP = 50000
N_RAMPUP = 100000
EPS = 1e-3

NC = 2
NS = 16
NW = NC * NS
L = 16

FPW = N_FACES // NW
PPW = N_CLOTH // NW

_MESH = dict(core_axis_name="c", subcore_axis_name="s", num_cores=NC,
             num_subcores=NS)


def _wid():
    return lax.axis_index("s") * NC + lax.axis_index("c")


def _stage_faces(opx, opy, opz, otx, oty, otz, f0, f1, f2, interpret=False):
    fvec = jax.ShapeDtypeStruct((N_FACES,), jnp.float32)

    @functools.partial(
        pl.kernel,
        out_type=[fvec] * 9,
        mesh=plsc.VectorSubcoreMesh(**_MESH),
        scratch_types=[
            pltpu.VMEM((N_VERTS,), jnp.float32),
            pltpu.VMEM((N_VERTS,), jnp.float32),
            pltpu.VMEM((N_VERTS,), jnp.float32),
            pltpu.VMEM((N_VERTS,), jnp.float32),
            pltpu.VMEM((N_VERTS,), jnp.float32),
            pltpu.VMEM((N_VERTS,), jnp.float32),
            pltpu.VMEM((FPW,), jnp.int32),
            pltpu.VMEM((FPW,), jnp.int32),
            pltpu.VMEM((FPW,), jnp.int32),
        ] + [pltpu.VMEM((FPW,), jnp.float32)] * 9,
        compiler_params=pltpu.CompilerParams(needs_layout_passes=False),
        interpret=interpret,
    )
    def body(opx_h, opy_h, opz_h, otx_h, oty_h, otz_h, f0_h, f1_h, f2_h,
             fcx_h, fcy_h, fcz_h, fsq_h, nx_h, ny_h, nz_h, nsq_h, braw_h,
             opx_v, opy_v, opz_v, otx_v, oty_v, otz_v, f0_v, f1_v, f2_v,
             fcx_v, fcy_v, fcz_v, fsq_v, nx_v, ny_v, nz_v, nsq_v, braw_v):
        base = _wid() * FPW
        pltpu.sync_copy(opx_h, opx_v)
        pltpu.sync_copy(opy_h, opy_v)
        pltpu.sync_copy(opz_h, opz_v)
        pltpu.sync_copy(otx_h, otx_v)
        pltpu.sync_copy(oty_h, oty_v)
        pltpu.sync_copy(otz_h, otz_v)
        pltpu.sync_copy(f0_h.at[pl.ds(base, FPW)], f0_v)
        pltpu.sync_copy(f1_h.at[pl.ds(base, FPW)], f1_v)
        pltpu.sync_copy(f2_h.at[pl.ds(base, FPW)], f2_v)
        for i in range(FPW // L):
            sl = pl.ds(i * L, L)
            i0 = f0_v[sl]
            i1 = f1_v[sl]
            i2 = f2_v[sl]
            ax = plsc.load_gather(opx_v, [i0])
            bx = plsc.load_gather(opx_v, [i1])
            cx = plsc.load_gather(opx_v, [i2])
            ay = plsc.load_gather(opy_v, [i0])
            by = plsc.load_gather(opy_v, [i1])
            cy = plsc.load_gather(opy_v, [i2])
            az = plsc.load_gather(opz_v, [i0])
            bz = plsc.load_gather(opz_v, [i1])
            cz = plsc.load_gather(opz_v, [i2])
            third = jnp.float32(1.0 / 3.0)
            fx = (ax + bx + cx) * third
            fy = (ay + by + cy) * third
            fz = (az + bz + cz) * third
            fcx_v[sl] = -2.0 * fx
            fcy_v[sl] = -2.0 * fy
            fcz_v[sl] = -2.0 * fz
            fsq_v[sl] = fx * fx + fy * fy + fz * fz
            tax = plsc.load_gather(otx_v, [i0])
            tbx = plsc.load_gather(otx_v, [i1])
            tcx = plsc.load_gather(otx_v, [i2])
            tay = plsc.load_gather(oty_v, [i0])
            tby = plsc.load_gather(oty_v, [i1])
            tcy = plsc.load_gather(oty_v, [i2])
            taz = plsc.load_gather(otz_v, [i0])
            tbz = plsc.load_gather(otz_v, [i1])
            tcz = plsc.load_gather(otz_v, [i2])
            px = (tax + tbx + tcx) * third
            py = (tay + tby + tcy) * third
            pz = (taz + tbz + tcz) * third
            e1x = tbx - tax
            e1y = tby - tay
            e1z = tbz - taz
            e2x = tcx - tax
            e2y = tcy - tay
            e2z = tcz - taz
            nx = e1y * e2z - e1z * e2y
            ny = e1z * e2x - e1x * e2z
            nz = e1x * e2y - e1y * e2x
            nx_v[sl] = nx
            ny_v[sl] = ny
            nz_v[sl] = nz
            nsq_v[sl] = nx * nx + ny * ny + nz * nz
            braw_v[sl] = px * nx + py * ny + pz * nz
        out_sl = pl.ds(base, FPW)
        pltpu.sync_copy(fcx_v, fcx_h.at[out_sl])
        pltpu.sync_copy(fcy_v, fcy_h.at[out_sl])
        pltpu.sync_copy(fcz_v, fcz_h.at[out_sl])
        pltpu.sync_copy(fsq_v, fsq_h.at[out_sl])
        pltpu.sync_copy(nx_v, nx_h.at[out_sl])
        pltpu.sync_copy(ny_v, ny_h.at[out_sl])
        pltpu.sync_copy(nz_v, nz_h.at[out_sl])
        pltpu.sync_copy(nsq_v, nsq_h.at[out_sl])
        pltpu.sync_copy(braw_v, braw_h.at[out_sl])

    return body(opx, opy, opz, otx, oty, otz, f0, f1, f2)


_ROWS = 256
_CHUNK = 1024


def _stage_argmin(c4, fp, interpret=False):
    def body(c4_ref, fp_ref, idx_ref):
        cb = c4_ref[...].astype(jnp.bfloat16)
        jloc = lax.broadcasted_iota(
            jnp.int32, (_ROWS, _CHUNK), 1).astype(jnp.float32)
        acc_min = jnp.full((_ROWS, 1), jnp.inf, dtype=jnp.float32)
        acc_idx = jnp.zeros((_ROWS, 1), dtype=jnp.float32)
        for jt in range(N_FACES // _CHUNK):
            f = fp_ref[:, jt * _CHUNK:(jt + 1) * _CHUNK]
            fsq = f[3:4, :]
            mm = lax.dot_general(
                cb, f.astype(jnp.bfloat16), (((1,), (0,)), ((), ())),
                preferred_element_type=jnp.float32,
            )
            s = fsq + mm
            tmin = jnp.min(s, axis=1, keepdims=True)
            tidx = jnp.min(
                jnp.where(s <= tmin, jloc, jnp.float32(1e30)),
                axis=1, keepdims=True) + jnp.float32(jt * _CHUNK)
            upd = tmin < acc_min
            acc_idx = jnp.where(upd, tidx, acc_idx)
            acc_min = jnp.where(upd, tmin, acc_min)
        idx_ref[...] = acc_idx.astype(jnp.int32)

    return pl.pallas_call(
        body,
        grid=(N_CLOTH // _ROWS,),
        in_specs=[
            pl.BlockSpec((_ROWS, 8), lambda i: (i, 0)),
            pl.BlockSpec((8, N_FACES), lambda i: (0, 0)),
        ],
        out_specs=pl.BlockSpec((_ROWS, 1), lambda i: (i, 0)),
        out_shape=jax.ShapeDtypeStruct((N_CLOTH, 1), jnp.int32),
        interpret=interpret,
    )(c4, fp)


def _stage_loss(nn_idx, nx, ny, nz, nsq, braw, ppx, ppy, ppz,
                interpret=False):
    @functools.partial(
        pl.kernel,
        out_type=jax.ShapeDtypeStruct((NW, L), jnp.float32),
        mesh=plsc.VectorSubcoreMesh(**_MESH),
        scratch_types=[
            pltpu.VMEM((N_FACES,), jnp.float32),
            pltpu.VMEM((N_FACES,), jnp.float32),
            pltpu.VMEM((N_FACES,), jnp.float32),
            pltpu.VMEM((N_FACES,), jnp.float32),
            pltpu.VMEM((N_FACES,), jnp.float32),
            pltpu.VMEM((PPW,), jnp.int32),
            pltpu.VMEM((PPW,), jnp.float32),
            pltpu.VMEM((PPW,), jnp.float32),
            pltpu.VMEM((PPW,), jnp.float32),
            pltpu.VMEM((L,), jnp.float32),
        ],
        compiler_params=pltpu.CompilerParams(needs_layout_passes=False),
        interpret=interpret,
    )
    def body(idx_h, nx_h, ny_h, nz_h, nsq_h, braw_h, ppx_h, ppy_h, ppz_h,
             out_h,
             nx_v, ny_v, nz_v, nsq_v, braw_v, idx_v, ppx_v, ppy_v, ppz_v,
             acc_v):
        wid = _wid()
        base = wid * PPW
        pltpu.sync_copy(nx_h, nx_v)
        pltpu.sync_copy(ny_h, ny_v)
        pltpu.sync_copy(nz_h, nz_v)
        pltpu.sync_copy(nsq_h, nsq_v)
        pltpu.sync_copy(braw_h, braw_v)
        pltpu.sync_copy(idx_h.at[pl.ds(base, PPW)], idx_v)
        pltpu.sync_copy(ppx_h.at[pl.ds(base, PPW)], ppx_v)
        pltpu.sync_copy(ppy_h.at[pl.ds(base, PPW)], ppy_v)
        pltpu.sync_copy(ppz_h.at[pl.ds(base, PPW)], ppz_v)
        acc = jnp.zeros((L,), jnp.float32)
        for i in range(PPW // L):
            sl = pl.ds(i * L, L)
            ii = idx_v[sl]
            gx = plsc.load_gather(nx_v, [ii])
            gy = plsc.load_gather(ny_v, [ii])
            gz = plsc.load_gather(nz_v, [ii])
            gq = plsc.load_gather(nsq_v, [ii])
            gb = plsc.load_gather(braw_v, [ii])
            draw = ppx_v[sl] * gx + ppy_v[sl] * gy + ppz_v[sl] * gz - gb
            xc = jnp.maximum(gq, jnp.float32(1e-36))
            y = plsc.bitcast(
                jnp.int32(0x5F3759DF) - (plsc.bitcast(xc, jnp.int32) >> 1),
                jnp.float32)
            for _ in range(3):
                y = y * (1.5 - 0.5 * xc * y * y)
            snorm = gq * y
            dist = draw / (snorm + jnp.float32(1e-12))
            t = jnp.maximum(jnp.float32(EPS) - dist, 0.0)
            acc = acc + t * t * t
        acc_v[...] = acc
        pltpu.sync_copy(acc_v, out_h.at[wid])

    return body(nn_idx, nx, ny, nz, nsq, braw, ppx, ppy, ppz)


def kernel(cloth_pos, cloth_pred_pos, obstacle_pos, obstacle_target_pos,
           obstacle_faces, iter_num):
    opx, opy, opz = (obstacle_pos[:, k] for k in range(3))
    otx, oty, otz = (obstacle_target_pos[:, k] for k in range(3))
    f0, f1, f2 = (obstacle_faces[:, k] for k in range(3))

    fcx, fcy, fcz, fsq, nx, ny, nz, nsq, braw = _stage_faces(
        opx, opy, opz, otx, oty, otz, f0, f1, f2)

    c4 = jnp.concatenate(
        [cloth_pos, jnp.zeros((N_CLOTH, 5), jnp.float32)], axis=1)
    fp = jnp.concatenate(
        [fcx[None], fcy[None], fcz[None], fsq[None],
         jnp.zeros((4, N_FACES), jnp.float32)], axis=0)
    nn_idx = _stage_argmin(c4, fp)[:, 0]

    ppx, ppy, ppz = (cloth_pred_pos[:, k] for k in range(3))
    partials = _stage_loss(nn_idx, nx, ny, nz, nsq, braw, ppx, ppy, ppz)

    it = jnp.maximum(iter_num - START_RAMPUP, 0)
    progress = jnp.minimum(it / N_RAMPUP, 1.0)
    weight = (WEIGHT_START + (WEIGHT_MAX - WEIGHT_START) * progress)
    return jnp.sum(partials) * weight.astype(jnp.float32)


# speedup vs baseline: 3.7166x; 1.0042x over previous
import functools

import jax
import jax.numpy as jnp
from jax import lax
from jax.experimental import pallas as pl
from jax.experimental.pallas import tpu as pltpu
from jax.experimental.pallas import tpu_sc as plsc

N_CLOTH = 8192
N_VERTS = 8192
N_FACES = 8192

WEIGHT_START = 1.0
WEIGHT_MAX = 5000.0
START_RAMPUP = 50000
N_RAMPUP = 100000
EPS = 1e-3

NC = 2
NS = 16
NW = NC * NS
L = 16

FPW = N_FACES // NW
PPW = N_CLOTH // NW

_MESH = dict(core_axis_name="c", subcore_axis_name="s", num_cores=NC,
             num_subcores=NS)


def _wid():
    return lax.axis_index("s") * NC + lax.axis_index("c")


def _stage_faces(opx, opy, opz, otx, oty, otz, f0, f1, f2, interpret=False):
    fvec = jax.ShapeDtypeStruct((N_FACES,), jnp.float32)

    @functools.partial(
        pl.kernel,
        out_type=[fvec] * 9,
        mesh=plsc.VectorSubcoreMesh(**_MESH),
        scratch_types=[
            pltpu.VMEM((N_VERTS,), jnp.float32),
            pltpu.VMEM((N_VERTS,), jnp.float32),
            pltpu.VMEM((N_VERTS,), jnp.float32),
            pltpu.VMEM((N_VERTS,), jnp.float32),
            pltpu.VMEM((N_VERTS,), jnp.float32),
            pltpu.VMEM((N_VERTS,), jnp.float32),
            pltpu.VMEM((FPW,), jnp.int32),
            pltpu.VMEM((FPW,), jnp.int32),
            pltpu.VMEM((FPW,), jnp.int32),
        ] + [pltpu.VMEM((FPW,), jnp.float32)] * 9,
        compiler_params=pltpu.CompilerParams(needs_layout_passes=False),
        interpret=interpret,
    )
    def body(opx_h, opy_h, opz_h, otx_h, oty_h, otz_h, f0_h, f1_h, f2_h,
             fcx_h, fcy_h, fcz_h, fsq_h, nx_h, ny_h, nz_h, nsq_h, braw_h,
             opx_v, opy_v, opz_v, otx_v, oty_v, otz_v, f0_v, f1_v, f2_v,
             fcx_v, fcy_v, fcz_v, fsq_v, nx_v, ny_v, nz_v, nsq_v, braw_v):
        base = _wid() * FPW
        pltpu.sync_copy(opx_h, opx_v)
        pltpu.sync_copy(opy_h, opy_v)
        pltpu.sync_copy(opz_h, opz_v)
        pltpu.sync_copy(otx_h, otx_v)
        pltpu.sync_copy(oty_h, oty_v)
        pltpu.sync_copy(otz_h, otz_v)
        pltpu.sync_copy(f0_h.at[pl.ds(base, FPW)], f0_v)
        pltpu.sync_copy(f1_h.at[pl.ds(base, FPW)], f1_v)
        pltpu.sync_copy(f2_h.at[pl.ds(base, FPW)], f2_v)
        for i in range(FPW // L):
            sl = pl.ds(i * L, L)
            i0 = f0_v[sl]
            i1 = f1_v[sl]
            i2 = f2_v[sl]
            ax = plsc.load_gather(opx_v, [i0])
            bx = plsc.load_gather(opx_v, [i1])
            cx = plsc.load_gather(opx_v, [i2])
            ay = plsc.load_gather(opy_v, [i0])
            by = plsc.load_gather(opy_v, [i1])
            cy = plsc.load_gather(opy_v, [i2])
            az = plsc.load_gather(opz_v, [i0])
            bz = plsc.load_gather(opz_v, [i1])
            cz = plsc.load_gather(opz_v, [i2])
            third = jnp.float32(1.0 / 3.0)
            fx = (ax + bx + cx) * third
            fy = (ay + by + cy) * third
            fz = (az + bz + cz) * third
            fcx_v[sl] = -2.0 * fx
            fcy_v[sl] = -2.0 * fy
            fcz_v[sl] = -2.0 * fz
            fsq_v[sl] = fx * fx + fy * fy + fz * fz
            tax = plsc.load_gather(otx_v, [i0])
            tbx = plsc.load_gather(otx_v, [i1])
            tcx = plsc.load_gather(otx_v, [i2])
            tay = plsc.load_gather(oty_v, [i0])
            tby = plsc.load_gather(oty_v, [i1])
            tcy = plsc.load_gather(oty_v, [i2])
            taz = plsc.load_gather(otz_v, [i0])
            tbz = plsc.load_gather(otz_v, [i1])
            tcz = plsc.load_gather(otz_v, [i2])
            px = (tax + tbx + tcx) * third
            py = (tay + tby + tcy) * third
            pz = (taz + tbz + tcz) * third
            e1x = tbx - tax
            e1y = tby - tay
            e1z = tbz - taz
            e2x = tcx - tax
            e2y = tcy - tay
            e2z = tcz - taz
            nx = e1y * e2z - e1z * e2y
            ny = e1z * e2x - e1x * e2z
            nz = e1x * e2y - e1y * e2x
            nx_v[sl] = nx
            ny_v[sl] = ny
            nz_v[sl] = nz
            nsq_v[sl] = nx * nx + ny * ny + nz * nz
            braw_v[sl] = px * nx + py * ny + pz * nz
        out_sl = pl.ds(base, FPW)
        pltpu.sync_copy(fcx_v, fcx_h.at[out_sl])
        pltpu.sync_copy(fcy_v, fcy_h.at[out_sl])
        pltpu.sync_copy(fcz_v, fcz_h.at[out_sl])
        pltpu.sync_copy(fsq_v, fsq_h.at[out_sl])
        pltpu.sync_copy(nx_v, nx_h.at[out_sl])
        pltpu.sync_copy(ny_v, ny_h.at[out_sl])
        pltpu.sync_copy(nz_v, nz_h.at[out_sl])
        pltpu.sync_copy(nsq_v, nsq_h.at[out_sl])
        pltpu.sync_copy(braw_v, braw_h.at[out_sl])

    return body(opx, opy, opz, otx, oty, otz, f0, f1, f2)


_ROWS = 512
_CHUNK = 1024


def _stage_argmin(c4, fp, interpret=False):
    def body(c4_ref, fp_ref, idx_ref):
        cb = c4_ref[...].astype(jnp.bfloat16)
        jloc = lax.broadcasted_iota(
            jnp.int32, (_ROWS, _CHUNK), 1).astype(jnp.float32)
        acc_min = jnp.full((_ROWS, 1), jnp.inf, dtype=jnp.float32)
        acc_idx = jnp.zeros((_ROWS, 1), dtype=jnp.float32)
        for jt in range(N_FACES // _CHUNK):
            f = fp_ref[:, jt * _CHUNK:(jt + 1) * _CHUNK]
            fsq = f[3:4, :]
            mm = lax.dot_general(
                cb, f.astype(jnp.bfloat16), (((1,), (0,)), ((), ())),
                preferred_element_type=jnp.float32,
            )
            s = fsq + mm
            tmin = jnp.min(s, axis=1, keepdims=True)
            tidx = jnp.min(
                jnp.where(s <= tmin, jloc, jnp.float32(1e30)),
                axis=1, keepdims=True) + jnp.float32(jt * _CHUNK)
            upd = tmin < acc_min
            acc_idx = jnp.where(upd, tidx, acc_idx)
            acc_min = jnp.where(upd, tmin, acc_min)
        idx_ref[...] = acc_idx.astype(jnp.int32)

    return pl.pallas_call(
        body,
        grid=(N_CLOTH // _ROWS,),
        in_specs=[
            pl.BlockSpec((_ROWS, 8), lambda i: (i, 0)),
            pl.BlockSpec((8, N_FACES), lambda i: (0, 0)),
        ],
        out_specs=pl.BlockSpec((_ROWS, 1), lambda i: (i, 0)),
        out_shape=jax.ShapeDtypeStruct((N_CLOTH, 1), jnp.int32),
        interpret=interpret,
    )(c4, fp)


def _stage_loss(nn_idx, nx, ny, nz, nsq, braw, ppx, ppy, ppz,
                interpret=False):
    @functools.partial(
        pl.kernel,
        out_type=jax.ShapeDtypeStruct((NW, L), jnp.float32),
        mesh=plsc.VectorSubcoreMesh(**_MESH),
        scratch_types=[
            pltpu.VMEM((N_FACES,), jnp.float32),
            pltpu.VMEM((N_FACES,), jnp.float32),
            pltpu.VMEM((N_FACES,), jnp.float32),
            pltpu.VMEM((N_FACES,), jnp.float32),
            pltpu.VMEM((N_FACES,), jnp.float32),
            pltpu.VMEM((PPW,), jnp.int32),
            pltpu.VMEM((PPW,), jnp.float32),
            pltpu.VMEM((PPW,), jnp.float32),
            pltpu.VMEM((PPW,), jnp.float32),
            pltpu.VMEM((L,), jnp.float32),
        ],
        compiler_params=pltpu.CompilerParams(needs_layout_passes=False),
        interpret=interpret,
    )
    def body(idx_h, nx_h, ny_h, nz_h, nsq_h, braw_h, ppx_h, ppy_h, ppz_h,
             out_h,
             nx_v, ny_v, nz_v, nsq_v, braw_v, idx_v, ppx_v, ppy_v, ppz_v,
             acc_v):
        wid = _wid()
        base = wid * PPW
        pltpu.sync_copy(nx_h, nx_v)
        pltpu.sync_copy(ny_h, ny_v)
        pltpu.sync_copy(nz_h, nz_v)
        pltpu.sync_copy(nsq_h, nsq_v)
        pltpu.sync_copy(braw_h, braw_v)
        pltpu.sync_copy(idx_h.at[pl.ds(base, PPW)], idx_v)
        pltpu.sync_copy(ppx_h.at[pl.ds(base, PPW)], ppx_v)
        pltpu.sync_copy(ppy_h.at[pl.ds(base, PPW)], ppy_v)
        pltpu.sync_copy(ppz_h.at[pl.ds(base, PPW)], ppz_v)
        acc = jnp.zeros((L,), jnp.float32)
        for i in range(PPW // L):
            sl = pl.ds(i * L, L)
            ii = idx_v[sl]
            gx = plsc.load_gather(nx_v, [ii])
            gy = plsc.load_gather(ny_v, [ii])
            gz = plsc.load_gather(nz_v, [ii])
            gq = plsc.load_gather(nsq_v, [ii])
            gb = plsc.load_gather(braw_v, [ii])
            draw = ppx_v[sl] * gx + ppy_v[sl] * gy + ppz_v[sl] * gz - gb
            xc = jnp.maximum(gq, jnp.float32(1e-36))
            y = plsc.bitcast(
                jnp.int32(0x5F3759DF) - (plsc.bitcast(xc, jnp.int32) >> 1),
                jnp.float32)
            for _ in range(3):
                y = y * (1.5 - 0.5 * xc * y * y)
            snorm = gq * y
            dist = draw / (snorm + jnp.float32(1e-12))
            t = jnp.maximum(jnp.float32(EPS) - dist, 0.0)
            acc = acc + t * t * t
        acc_v[...] = acc
        pltpu.sync_copy(acc_v, out_h.at[wid])

    return body(nn_idx, nx, ny, nz, nsq, braw, ppx, ppy, ppz)


def kernel(cloth_pos, cloth_pred_pos, obstacle_pos, obstacle_target_pos,
           obstacle_faces, iter_num):
    opx, opy, opz = (obstacle_pos[:, k] for k in range(3))
    otx, oty, otz = (obstacle_target_pos[:, k] for k in range(3))
    f0, f1, f2 = (obstacle_faces[:, k] for k in range(3))

    fcx, fcy, fcz, fsq, nx, ny, nz, nsq, braw = _stage_faces(
        opx, opy, opz, otx, oty, otz, f0, f1, f2)

    c4 = jnp.concatenate(
        [cloth_pos, jnp.zeros((N_CLOTH, 5), jnp.float32)], axis=1)
    fp = jnp.concatenate(
        [fcx[None], fcy[None], fcz[None], fsq[None],
         jnp.zeros((4, N_FACES), jnp.float32)], axis=0)
    nn_idx = _stage_argmin(c4, fp)[:, 0]

    ppx, ppy, ppz = (cloth_pred_pos[:, k] for k in range(3))
    partials = _stage_loss(nn_idx, nx, ny, nz, nsq, braw, ppx, ppy, ppz)

    it = jnp.maximum(iter_num - START_RAMPUP, 0)
    progress = jnp.minimum(it / N_RAMPUP, 1.0)
    weight = (WEIGHT_START + (WEIGHT_MAX - WEIGHT_START) * progress)
    return jnp.sum(partials) * weight.astype(jnp.float32)


# speedup vs baseline: 3.9784x; 1.0704x over previous
import functools

import jax
import jax.numpy as jnp
from jax import lax
from jax.experimental import pallas as pl
from jax.experimental.pallas import tpu as pltpu
from jax.experimental.pallas import tpu_sc as plsc

N_CLOTH = 8192
N_VERTS = 8192
N_FACES = 8192

WEIGHT_START = 1.0
WEIGHT_MAX = 5000.0
START_RAMPUP = 50000
N_RAMPUP = 100000
EPS = 1e-3

NC = 2
NS = 16
NW = NC * NS
L = 16

FPW = N_FACES // NW
PPW = N_CLOTH // NW

_MESH = dict(core_axis_name="c", subcore_axis_name="s", num_cores=NC,
             num_subcores=NS)


def _wid():
    return lax.axis_index("s") * NC + lax.axis_index("c")


def _stage_faces(opx, opy, opz, otx, oty, otz, f0, f1, f2, interpret=False):
    fvec = jax.ShapeDtypeStruct((N_FACES,), jnp.float32)

    @functools.partial(
        pl.kernel,
        out_type=[fvec] * 9,
        mesh=plsc.VectorSubcoreMesh(**_MESH),
        scratch_types=[
            pltpu.VMEM((N_VERTS,), jnp.float32),
            pltpu.VMEM((N_VERTS,), jnp.float32),
            pltpu.VMEM((N_VERTS,), jnp.float32),
            pltpu.VMEM((N_VERTS,), jnp.float32),
            pltpu.VMEM((N_VERTS,), jnp.float32),
            pltpu.VMEM((N_VERTS,), jnp.float32),
            pltpu.VMEM((FPW,), jnp.int32),
            pltpu.VMEM((FPW,), jnp.int32),
            pltpu.VMEM((FPW,), jnp.int32),
        ] + [pltpu.VMEM((FPW,), jnp.float32)] * 9,
        compiler_params=pltpu.CompilerParams(needs_layout_passes=False),
        interpret=interpret,
    )
    def body(opx_h, opy_h, opz_h, otx_h, oty_h, otz_h, f0_h, f1_h, f2_h,
             fcx_h, fcy_h, fcz_h, fsq_h, nx_h, ny_h, nz_h, nsq_h, braw_h,
             opx_v, opy_v, opz_v, otx_v, oty_v, otz_v, f0_v, f1_v, f2_v,
             fcx_v, fcy_v, fcz_v, fsq_v, nx_v, ny_v, nz_v, nsq_v, braw_v):
        base = _wid() * FPW
        pltpu.sync_copy(opx_h, opx_v)
        pltpu.sync_copy(opy_h, opy_v)
        pltpu.sync_copy(opz_h, opz_v)
        pltpu.sync_copy(otx_h, otx_v)
        pltpu.sync_copy(oty_h, oty_v)
        pltpu.sync_copy(otz_h, otz_v)
        pltpu.sync_copy(f0_h.at[pl.ds(base, FPW)], f0_v)
        pltpu.sync_copy(f1_h.at[pl.ds(base, FPW)], f1_v)
        pltpu.sync_copy(f2_h.at[pl.ds(base, FPW)], f2_v)
        for i in range(FPW // L):
            sl = pl.ds(i * L, L)
            i0 = f0_v[sl]
            i1 = f1_v[sl]
            i2 = f2_v[sl]
            ax = plsc.load_gather(opx_v, [i0])
            bx = plsc.load_gather(opx_v, [i1])
            cx = plsc.load_gather(opx_v, [i2])
            ay = plsc.load_gather(opy_v, [i0])
            by = plsc.load_gather(opy_v, [i1])
            cy = plsc.load_gather(opy_v, [i2])
            az = plsc.load_gather(opz_v, [i0])
            bz = plsc.load_gather(opz_v, [i1])
            cz = plsc.load_gather(opz_v, [i2])
            third = jnp.float32(1.0 / 3.0)
            fx = (ax + bx + cx) * third
            fy = (ay + by + cy) * third
            fz = (az + bz + cz) * third
            fcx_v[sl] = -2.0 * fx
            fcy_v[sl] = -2.0 * fy
            fcz_v[sl] = -2.0 * fz
            fsq_v[sl] = fx * fx + fy * fy + fz * fz
            tax = plsc.load_gather(otx_v, [i0])
            tbx = plsc.load_gather(otx_v, [i1])
            tcx = plsc.load_gather(otx_v, [i2])
            tay = plsc.load_gather(oty_v, [i0])
            tby = plsc.load_gather(oty_v, [i1])
            tcy = plsc.load_gather(oty_v, [i2])
            taz = plsc.load_gather(otz_v, [i0])
            tbz = plsc.load_gather(otz_v, [i1])
            tcz = plsc.load_gather(otz_v, [i2])
            px = (tax + tbx + tcx) * third
            py = (tay + tby + tcy) * third
            pz = (taz + tbz + tcz) * third
            e1x = tbx - tax
            e1y = tby - tay
            e1z = tbz - taz
            e2x = tcx - tax
            e2y = tcy - tay
            e2z = tcz - taz
            nx = e1y * e2z - e1z * e2y
            ny = e1z * e2x - e1x * e2z
            nz = e1x * e2y - e1y * e2x
            nx_v[sl] = nx
            ny_v[sl] = ny
            nz_v[sl] = nz
            nsq_v[sl] = nx * nx + ny * ny + nz * nz
            braw_v[sl] = px * nx + py * ny + pz * nz
        out_sl = pl.ds(base, FPW)
        pltpu.sync_copy(fcx_v, fcx_h.at[out_sl])
        pltpu.sync_copy(fcy_v, fcy_h.at[out_sl])
        pltpu.sync_copy(fcz_v, fcz_h.at[out_sl])
        pltpu.sync_copy(fsq_v, fsq_h.at[out_sl])
        pltpu.sync_copy(nx_v, nx_h.at[out_sl])
        pltpu.sync_copy(ny_v, ny_h.at[out_sl])
        pltpu.sync_copy(nz_v, nz_h.at[out_sl])
        pltpu.sync_copy(nsq_v, nsq_h.at[out_sl])
        pltpu.sync_copy(braw_v, braw_h.at[out_sl])

    return body(opx, opy, opz, otx, oty, otz, f0, f1, f2)


_ROWS = 512
_CHUNK = 1024


def _stage_argmin(c4, fp, interpret=False):
    def body(c4_ref, fp_ref, idx_ref):
        cb = c4_ref[...].astype(jnp.bfloat16)
        jloc = lax.broadcasted_iota(
            jnp.int32, (_ROWS, _CHUNK), 1).astype(jnp.float32)
        acc_min = jnp.full((_ROWS, 1), jnp.inf, dtype=jnp.float32)
        acc_idx = jnp.zeros((_ROWS, 1), dtype=jnp.float32)
        for jt in range(N_FACES // _CHUNK):
            f = fp_ref[:, jt * _CHUNK:(jt + 1) * _CHUNK]
            s = lax.dot_general(
                cb, f.astype(jnp.bfloat16), (((1,), (0,)), ((), ())),
                preferred_element_type=jnp.float32,
            )
            tmin = jnp.min(s, axis=1, keepdims=True)
            tidx = jnp.min(
                jnp.where(s <= tmin, jloc, jnp.float32(1e30)),
                axis=1, keepdims=True) + jnp.float32(jt * _CHUNK)
            upd = tmin < acc_min
            acc_idx = jnp.where(upd, tidx, acc_idx)
            acc_min = jnp.where(upd, tmin, acc_min)
        idx_ref[...] = acc_idx.astype(jnp.int32)

    return pl.pallas_call(
        body,
        grid=(N_CLOTH // _ROWS,),
        in_specs=[
            pl.BlockSpec((_ROWS, 8), lambda i: (i, 0)),
            pl.BlockSpec((8, N_FACES), lambda i: (0, 0)),
        ],
        out_specs=pl.BlockSpec((_ROWS, 1), lambda i: (i, 0)),
        out_shape=jax.ShapeDtypeStruct((N_CLOTH, 1), jnp.int32),
        interpret=interpret,
    )(c4, fp)


def _stage_loss(nn_idx, nx, ny, nz, nsq, braw, ppx, ppy, ppz,
                interpret=False):
    @functools.partial(
        pl.kernel,
        out_type=jax.ShapeDtypeStruct((NW, L), jnp.float32),
        mesh=plsc.VectorSubcoreMesh(**_MESH),
        scratch_types=[
            pltpu.VMEM((N_FACES,), jnp.float32),
            pltpu.VMEM((N_FACES,), jnp.float32),
            pltpu.VMEM((N_FACES,), jnp.float32),
            pltpu.VMEM((N_FACES,), jnp.float32),
            pltpu.VMEM((N_FACES,), jnp.float32),
            pltpu.VMEM((PPW,), jnp.int32),
            pltpu.VMEM((PPW,), jnp.float32),
            pltpu.VMEM((PPW,), jnp.float32),
            pltpu.VMEM((PPW,), jnp.float32),
            pltpu.VMEM((L,), jnp.float32),
        ],
        compiler_params=pltpu.CompilerParams(needs_layout_passes=False),
        interpret=interpret,
    )
    def body(idx_h, nx_h, ny_h, nz_h, nsq_h, braw_h, ppx_h, ppy_h, ppz_h,
             out_h,
             nx_v, ny_v, nz_v, nsq_v, braw_v, idx_v, ppx_v, ppy_v, ppz_v,
             acc_v):
        wid = _wid()
        base = wid * PPW
        pltpu.sync_copy(nx_h, nx_v)
        pltpu.sync_copy(ny_h, ny_v)
        pltpu.sync_copy(nz_h, nz_v)
        pltpu.sync_copy(nsq_h, nsq_v)
        pltpu.sync_copy(braw_h, braw_v)
        pltpu.sync_copy(idx_h.at[pl.ds(base, PPW)], idx_v)
        pltpu.sync_copy(ppx_h.at[pl.ds(base, PPW)], ppx_v)
        pltpu.sync_copy(ppy_h.at[pl.ds(base, PPW)], ppy_v)
        pltpu.sync_copy(ppz_h.at[pl.ds(base, PPW)], ppz_v)
        acc = jnp.zeros((L,), jnp.float32)
        for i in range(PPW // L):
            sl = pl.ds(i * L, L)
            ii = idx_v[sl]
            gx = plsc.load_gather(nx_v, [ii])
            gy = plsc.load_gather(ny_v, [ii])
            gz = plsc.load_gather(nz_v, [ii])
            gq = plsc.load_gather(nsq_v, [ii])
            gb = plsc.load_gather(braw_v, [ii])
            draw = ppx_v[sl] * gx + ppy_v[sl] * gy + ppz_v[sl] * gz - gb
            xc = jnp.maximum(gq, jnp.float32(1e-36))
            y = plsc.bitcast(
                jnp.int32(0x5F3759DF) - (plsc.bitcast(xc, jnp.int32) >> 1),
                jnp.float32)
            for _ in range(3):
                y = y * (1.5 - 0.5 * xc * y * y)
            snorm = gq * y
            dist = draw / (snorm + jnp.float32(1e-12))
            t = jnp.maximum(jnp.float32(EPS) - dist, 0.0)
            acc = acc + t * t * t
        acc_v[...] = acc
        pltpu.sync_copy(acc_v, out_h.at[wid])

    return body(nn_idx, nx, ny, nz, nsq, braw, ppx, ppy, ppz)


def kernel(cloth_pos, cloth_pred_pos, obstacle_pos, obstacle_target_pos,
           obstacle_faces, iter_num):
    opx, opy, opz = (obstacle_pos[:, k] for k in range(3))
    otx, oty, otz = (obstacle_target_pos[:, k] for k in range(3))
    f0, f1, f2 = (obstacle_faces[:, k] for k in range(3))

    fcx, fcy, fcz, fsq, nx, ny, nz, nsq, braw = _stage_faces(
        opx, opy, opz, otx, oty, otz, f0, f1, f2)

    c4 = jnp.concatenate(
        [cloth_pos,
         jnp.ones((N_CLOTH, 3), jnp.float32),
         jnp.zeros((N_CLOTH, 2), jnp.float32)], axis=1)
    b1 = fsq.astype(jnp.bfloat16).astype(jnp.float32)
    r1 = fsq - b1
    b2 = r1.astype(jnp.bfloat16).astype(jnp.float32)
    b3 = (r1 - b2).astype(jnp.bfloat16).astype(jnp.float32)
    fp = jnp.concatenate(
        [fcx[None], fcy[None], fcz[None], b1[None], b2[None], b3[None],
         jnp.zeros((2, N_FACES), jnp.float32)], axis=0)
    nn_idx = _stage_argmin(c4, fp)[:, 0]

    ppx, ppy, ppz = (cloth_pred_pos[:, k] for k in range(3))
    partials = _stage_loss(nn_idx, nx, ny, nz, nsq, braw, ppx, ppy, ppz)

    it = jnp.maximum(iter_num - START_RAMPUP, 0)
    progress = jnp.minimum(it / N_RAMPUP, 1.0)
    weight = (WEIGHT_START + (WEIGHT_MAX - WEIGHT_START) * progress)
    return jnp.sum(partials) * weight.astype(jnp.float32)
